# baseline (speedup 1.0000x reference)
import functools

import jax
import jax.numpy as jnp
from jax import lax
from jax.experimental import pallas as pl
from jax.experimental.pallas import tpu as pltpu
from jax.experimental.pallas import tpu_sc as plsc

NC, NS, L = 2, 16, 16
NW = NC * NS

NROW, NCOL, D = 16384, 200, 4
NVOC = 10
JB, JS = 25, 8
IB, IL = 128, 128
IB_PER_W = IB // NW
IBQ = IB_PER_W // 2
NGRP = IL // L

_GATHER_MODE = lax.GatherScatterMode.PROMISE_IN_BOUNDS


def _sc_body(x_hbm, wt_hbm, out_hbm, xbuf, obuf, wt_v,
             insem0, insem1, outsem0, outsem1):
    insems = (insem0, insem1)
    outsems = (outsem0, outsem1)
    wid = lax.axis_index("s") * NC + lax.axis_index("c")
    ib0 = wid * IB_PER_W

    pltpu.sync_copy(wt_hbm, wt_v)
    lane = jnp.minimum(lax.iota(jnp.int32, L), NVOC - 1)
    wcols = tuple(
        plsc.load_gather(wt_v, [jnp.full((L,), c, jnp.int32), lane])
        for c in range(4)
    )

    def in_pair(jb, b):
        return (x_hbm.at[jb, pl.ds(ib0, IB_PER_W)],
                xbuf.at[b], insems[b])

    def out_pair(jb, b):
        return (obuf.at[b],
                out_hbm.at[pl.ds(jb * JS, JS), pl.ds(ib0, IB_PER_W)],
                outsems[b])

    def compute(b):
        for js in range(JS):
            for ib in range(IB_PER_W):
                for k in range(NGRP):
                    idx = xbuf[b, ib, js, pl.ds(k * L, L)]
                    for c, w in enumerate(wcols):
                        vals = jnp.take_along_axis(
                            w, idx, axis=0, mode=_GATHER_MODE)
                        obuf[b, js, ib, c, pl.ds(k * L, L)] = vals

    pltpu.async_copy(*in_pair(0, 0))
    pltpu.async_copy(*in_pair(1, 1))

    def pair_iter(p, carry):
        for b in range(2):
            jb = p * 2 + b
            pltpu.make_async_copy(*in_pair(jb, b)).wait()

            @pl.when(jb >= 2)
            def _wait_prev_out():
                pltpu.make_async_copy(*out_pair(jb - 2, b)).wait()

            compute(b)
            pltpu.async_copy(*out_pair(jb, b))

            @pl.when(jb + 2 < JB)
            def _prefetch_next():
                pltpu.async_copy(*in_pair(jb + 2, b))
        return carry

    lax.fori_loop(0, JB // 2, pair_iter, 0)
    pltpu.make_async_copy(*in_pair(JB - 1, 0)).wait()
    pltpu.make_async_copy(*out_pair(JB - 3, 0)).wait()
    compute(0)
    pltpu.async_copy(*out_pair(JB - 1, 0))
    pltpu.make_async_copy(*out_pair(JB - 2, 1)).wait()
    pltpu.make_async_copy(*out_pair(JB - 1, 0)).wait()


@functools.partial(
    pl.kernel,
    out_type=jax.ShapeDtypeStruct((NCOL, IB, D, IL), jnp.float32),
    mesh=plsc.VectorSubcoreMesh(core_axis_name="c", subcore_axis_name="s"),
    compiler_params=pltpu.CompilerParams(needs_layout_passes=False),
    scratch_types=[
        pltpu.VMEM((2, IB_PER_W, JS, IL), jnp.int32),
        pltpu.VMEM((2, JS, IB_PER_W, D, IL), jnp.float32),
        pltpu.VMEM((4, NVOC), jnp.float32),
        pltpu.SemaphoreType.DMA,
        pltpu.SemaphoreType.DMA,
        pltpu.SemaphoreType.DMA,
        pltpu.SemaphoreType.DMA,
    ],
)
def _lookup(x_hbm, wt_hbm, out_hbm, xbuf, obuf, wt_v,
            insem0, insem1, outsem0, outsem1):
    _sc_body(x_hbm, wt_hbm, out_hbm, xbuf, obuf, wt_v,
             insem0, insem1, outsem0, outsem1)


def kernel(x, W):
    wt = W.T
    xr = x.T.reshape(JB, JS, IB, IL).transpose(0, 2, 1, 3)
    out4 = _lookup(xr, wt)
    return out4.transpose(1, 3, 0, 2).reshape(NROW, NCOL, D)


# speedup vs baseline: 1.2340x; 1.2340x over previous
import functools

import jax
import jax.numpy as jnp
from jax import lax
from jax.experimental import pallas as pl
from jax.experimental.pallas import tpu as pltpu
from jax.experimental.pallas import tpu_sc as plsc

NC, NS, L = 2, 16, 16
NW = NC * NS

NROW, NCOL, D = 16384, 200, 4
NVOC = 10
JB, JS = 25, 8
IB, IL = 128, 128
IB_PER_W = IB // NW
IBQ = IB_PER_W // 2
NGRP = IL // L

_GATHER_MODE = lax.GatherScatterMode.PROMISE_IN_BOUNDS


def _sc_body(x_hbm, wt_hbm, out_hbm, xbuf, obuf, wt_v,
             insem0, insem1, outsem0, outsem1):
    insems = (insem0, insem1)
    outsems = (outsem0, outsem1)
    wid = lax.axis_index("s") * NC + lax.axis_index("c")
    ib0 = wid * IB_PER_W

    pltpu.sync_copy(wt_hbm, wt_v)
    lane = jnp.minimum(lax.iota(jnp.int32, L), NVOC - 1)
    wcols = tuple(
        plsc.load_gather(wt_v, [jnp.full((L,), c, jnp.int32), lane])
        for c in range(4)
    )

    def in_pair(jb, b):
        return (x_hbm.at[jb, pl.ds(ib0 + b * IBQ, IBQ)],
                xbuf.at[b], insems[b])

    def out_pair(jb, b):
        return (obuf.at[b],
                out_hbm.at[pl.ds(jb * JS, JS), pl.ds(ib0 + b * IBQ, IBQ)],
                outsems[b])

    def compute(b):
        for js in range(JS):
            for ib in range(IBQ):
                for k in range(NGRP):
                    idx = xbuf[b, ib, js, pl.ds(k * L, L)]
                    for c, w in enumerate(wcols):
                        vals = jnp.take_along_axis(
                            w, idx, axis=0, mode=_GATHER_MODE)
                        obuf[b, js, ib, c, pl.ds(k * L, L)] = vals

    pltpu.async_copy(*in_pair(0, 0))
    pltpu.async_copy(*in_pair(0, 1))

    def jb_iter(jb, carry):
        for b in range(2):
            pltpu.make_async_copy(*in_pair(jb, b)).wait()

            @pl.when(jb >= 1)
            def _wait_prev_out():
                pltpu.make_async_copy(*out_pair(jb - 1, b)).wait()

            compute(b)
            pltpu.async_copy(*out_pair(jb, b))

            @pl.when(jb + 1 < JB)
            def _prefetch_next():
                pltpu.async_copy(*in_pair(jb + 1, b))
        return carry

    lax.fori_loop(0, JB, jb_iter, 0)
    pltpu.make_async_copy(*out_pair(JB - 1, 0)).wait()
    pltpu.make_async_copy(*out_pair(JB - 1, 1)).wait()


@functools.partial(
    pl.kernel,
    out_type=jax.ShapeDtypeStruct((NCOL, IB, D, IL), jnp.float32),
    mesh=plsc.VectorSubcoreMesh(core_axis_name="c", subcore_axis_name="s"),
    compiler_params=pltpu.CompilerParams(needs_layout_passes=False),
    scratch_types=[
        pltpu.VMEM((2, IBQ, JS, IL), jnp.int32),
        pltpu.VMEM((2, JS, IBQ, D, IL), jnp.float32),
        pltpu.VMEM((4, NVOC), jnp.float32),
        pltpu.SemaphoreType.DMA,
        pltpu.SemaphoreType.DMA,
        pltpu.SemaphoreType.DMA,
        pltpu.SemaphoreType.DMA,
    ],
)
def _lookup(x_hbm, wt_hbm, out_hbm, xbuf, obuf, wt_v,
            insem0, insem1, outsem0, outsem1):
    _sc_body(x_hbm, wt_hbm, out_hbm, xbuf, obuf, wt_v,
             insem0, insem1, outsem0, outsem1)


def kernel(x, W):
    wt = W.T
    xr = x.T.reshape(JB, JS, IB, IL).transpose(0, 2, 1, 3)
    out4 = _lookup(xr, wt)
    return out4.transpose(1, 3, 0, 2).reshape(NROW, NCOL, D)
